# Initial kernel scaffold; baseline (speedup 1.0000x reference)
import jax
import jax.numpy as jnp
from jax.experimental import pallas as pl


def kernel(sites, bonds, states, params, indices1, indices2):
    raise NotImplementedError("write your pallas kernel here")


# speedup vs baseline: 1.5039x; 1.5039x over previous
import functools

import jax
import jax.numpy as jnp
from jax import lax
from jax.experimental import pallas as pl
from jax.experimental.pallas import tpu as pltpu
from jax.experimental.pallas import tpu_sc as plsc

D = 128
_NC, _NS = 2, 16
_NW = _NC * _NS
_CHUNK = 80

_F32 = jnp.float32


def _relu(x):
    return jnp.maximum(x, 0.0)


def _dot(a, b):
    return jnp.dot(a, b, preferred_element_type=_F32)


def _full(shape):
    return pl.BlockSpec(shape, lambda i: (0,) * len(shape))


def _site_pre_body(x_ref, w1, b1, w2, b2, wa, wb, s1_ref, pa_ref, pb_ref):
    x = x_ref[...]
    h = _relu(_dot(x, w1[...]) + b1[...])
    s1 = _relu(_dot(h, w2[...]) + b2[...])
    s1_ref[...] = s1
    pa_ref[...] = _dot(s1, wa[...])
    pb_ref[...] = _dot(s1, wb[...])


def _state_pre_body(st, w1, b1, w2, b2, wd, bub1, scw, sub1,
                    s1_ref, g_ref, s_ref):
    s = st[...]
    h = _relu(_dot(s, w1[...]) + b1[...])
    s1 = _relu(_dot(h, w2[...]) + b2[...])
    s1_ref[...] = s1
    g_ref[...] = _dot(s1, wd[...]) + bub1[...]
    s_ref[...] = _dot(s1, scw[...]) + sub1[...]


def _bond_body(x_ref, g1_ref, g2_ref, w1, b1, w2, b2, wc, uw2, ub2, uw3, ub3,
               gconst, bo_ref, b2o_ref):
    x = x_ref[...]
    h = _relu(_dot(x, w1[...]) + b1[...])
    b1t = _relu(_dot(h, w2[...]) + b2[...])
    h = _relu(g1_ref[...] + g2_ref[...] + _dot(b1t, wc[...]) + gconst[...])
    h = _relu(_dot(h, uw2[...]) + ub2[...])
    b2v = _dot(h, uw3[...]) + ub3[...]
    bo_ref[...] = x + b2v
    b2o_ref[...] = b2v


def _site_upd_body(ps_ref, pc_ref, s1_ref, x_ref, wa, wb, sconst,
                   w2, b2, w3, b3, out_ref, bp_ref, sp_ref):
    i = pl.program_id(0)
    sums = ps_ref[...]
    cnt = pc_ref[:, 0:1]
    pool = sums / jnp.maximum(cnt, 1.0)
    h = _relu(_dot(pool, wa[...]) + _dot(s1_ref[...], wb[...]) + sconst[...])
    h = _relu(_dot(h, w2[...]) + b2[...])
    s2 = _relu(_dot(h, w3[...]) + b3[...])
    out_ref[...] = x_ref[...] + s2

    @pl.when(i == 0)
    def _init():
        bp_ref[...] = jnp.zeros_like(bp_ref)
        sp_ref[...] = jnp.zeros_like(sp_ref)

    bp_ref[...] += jnp.sum(sums, axis=0, keepdims=True)
    sp_ref[...] += jnp.sum(s2, axis=0, keepdims=True)


def _make_state_upd_body(e_count, n_count):
    def _state_upd_body(bp, sp, s1, st0, ta, tb, tcw, b1, w2, b2, w3, b3,
                        out_ref):
        bpv = bp[...] * (1.0 / e_count)
        spv = sp[...] * (1.0 / n_count)
        h = _relu(_dot(bpv, ta[...]) + _dot(spv, tb[...])
                  + _dot(s1[...], tcw[...]) + b1[...])
        h = _relu(_dot(h, w2[...]) + b2[...])
        s2 = _relu(_dot(h, w3[...]) + b3[...])
        out_ref[...] = st0[...] + s2
    return _state_upd_body


def _sc_gather(t1, t2, i1, i2):
    e_count = i1.shape[0]
    epw = e_count // _NW
    nit = epw // _CHUNK
    mesh = plsc.VectorSubcoreMesh(core_axis_name="c", subcore_axis_name="s")

    @functools.partial(
        pl.kernel, mesh=mesh,
        out_type=(jax.ShapeDtypeStruct((e_count, D), _F32),
                  jax.ShapeDtypeStruct((e_count, D), _F32)),
        scratch_types=[
            pltpu.VMEM((_CHUNK,), jnp.int32),
            pltpu.VMEM((_CHUNK,), jnp.int32),
            pltpu.VMEM((_CHUNK, D), _F32),
            pltpu.VMEM((_CHUNK, D), _F32),
            pltpu.SemaphoreType.DMA,
            pltpu.SemaphoreType.DMA,
        ],
    )
    def gather_k(t1h, t2h, i1h, i2h, g1h, g2h, i1v, i2v, r1v, r2v, sm1, sm2):
        wid = lax.axis_index("s") * _NC + lax.axis_index("c")
        base = wid * epw

        def body(j, carry):
            off = base + j * _CHUNK
            pltpu.sync_copy(i1h.at[pl.ds(off, _CHUNK)], i1v)
            pltpu.sync_copy(i2h.at[pl.ds(off, _CHUNK)], i2v)
            cp1 = pltpu.async_copy(t1h.at[i1v], r1v, sm1)
            cp2 = pltpu.async_copy(t2h.at[i2v], r2v, sm2)
            cp1.wait()
            cp2.wait()
            pltpu.sync_copy(r1v, g1h.at[pl.ds(off, _CHUNK)])
            pltpu.sync_copy(r2v, g2h.at[pl.ds(off, _CHUNK)])
            return carry

        lax.fori_loop(0, nit, body, 0)

    return gather_k(t1, t2, i1, i2)


def _sc_scatter(b2, i1, n_count):
    e_count = i1.shape[0]
    epw = e_count // _NS
    nit = epw // _CHUNK
    align = _NS * _CHUNK
    npad = ((n_count + align - 1) // align) * align
    rps = npad // _NS
    fpc = rps // _CHUNK
    mesh = plsc.VectorSubcoreMesh(core_axis_name="c", subcore_axis_name="s",
                                  num_cores=1)
    zd = jnp.zeros((_CHUNK, D), _F32)
    ones = jnp.ones((_CHUNK, D), _F32)

    @functools.partial(
        pl.kernel, mesh=mesh,
        out_type=(jax.ShapeDtypeStruct((npad, D), _F32),
                  jax.ShapeDtypeStruct((npad, D), _F32)),
        scratch_types=[
            pltpu.VMEM((_CHUNK,), jnp.int32),
            pltpu.VMEM((_CHUNK, D), _F32),
            pltpu.VMEM((_CHUNK, D), _F32),
            pltpu.VMEM((_CHUNK, D), _F32),
            pltpu.VMEM_SHARED((npad, D), _F32),
        ],
    )
    def scatter_k(b2h, i1h, zdh, onesh, osum, ocnt,
                  i1v, rv, zbuf, ones_v, ssum):
        sid = lax.axis_index("s")
        pltpu.sync_copy(zdh, zbuf)
        pltpu.sync_copy(onesh, ones_v)
        rbase = sid * rps

        def zero():
            def zb(t, carry):
                pltpu.sync_copy(zbuf,
                                ssum.at[pl.ds(rbase + t * _CHUNK, _CHUNK)])
                return carry

            lax.fori_loop(0, fpc, zb, 0)

        def flush(dsth):
            def fb(t, carry):
                srcr = rbase + t * _CHUNK
                pltpu.sync_copy(ssum.at[pl.ds(srcr, _CHUNK)], rv)
                pltpu.sync_copy(rv, dsth.at[pl.ds(srcr, _CHUNK)])
                return carry

            lax.fori_loop(0, fpc, fb, 0)

        base = sid * epw

        zero()
        plsc.subcore_barrier()

        def body(j, carry):
            off = base + j * _CHUNK
            pltpu.sync_copy(i1h.at[pl.ds(off, _CHUNK)], i1v)
            pltpu.sync_copy(b2h.at[pl.ds(off, _CHUNK)], rv)
            pltpu.sync_copy(rv, ssum.at[i1v], add=True)
            return carry

        lax.fori_loop(0, nit, body, 0)
        plsc.subcore_barrier()
        flush(osum)
        plsc.subcore_barrier()

        zero()
        plsc.subcore_barrier()

        def cbody(j, carry):
            off = base + j * _CHUNK
            pltpu.sync_copy(i1h.at[pl.ds(off, _CHUNK)], i1v)
            pltpu.sync_copy(ones_v, ssum.at[i1v], add=True)
            return carry

        lax.fori_loop(0, nit, cbody, 0)
        plsc.subcore_barrier()
        flush(ocnt)

    return scatter_k(b2, i1, zd, ones)


def kernel(sites, bonds, states, params, indices1, indices2):
    p = params
    xs = sites[0]
    xb = bonds[0]
    n_count, e_count = xs.shape[0], xb.shape[0]
    i1 = indices1.astype(jnp.int32)
    i2 = indices2.astype(jnp.int32)

    def b_(name):
        return p[name].reshape(1, D)

    wmat = _full((D, D))
    wrow = _full((1, D))
    bu_wa = p["bu_w1"][0:D]
    bu_wb = p["bu_w1"][D:2 * D]
    bu_wc = p["bu_w1"][2 * D:3 * D]
    bu_wd = p["bu_w1"][3 * D:]
    su_wa = p["su_w1"][0:D]
    su_wb = p["su_w1"][D:2 * D]
    su_wc = p["su_w1"][2 * D:]
    stu_wa = p["stu_w1"][0:D]
    stu_wb = p["stu_w1"][D:2 * D]
    stu_wc = p["stu_w1"][2 * D:]

    tn = 1000
    rown = pl.BlockSpec((tn, D), lambda i: (i, 0))
    sites1, s1p, s2p = pl.pallas_call(
        _site_pre_body,
        grid=(n_count // tn,),
        in_specs=[rown, wmat, wrow, wmat, wrow, wmat, wmat],
        out_specs=[rown, rown, rown],
        out_shape=[jax.ShapeDtypeStruct((n_count, D), _F32)] * 3,
    )(xs, p["sfc_w1"], b_("sfc_b1"), p["sfc_w2"], b_("sfc_b2"), bu_wa, bu_wb)

    states1, gconst, sconst = pl.pallas_call(
        _state_pre_body,
        grid=(1,),
        in_specs=[wrow, wmat, wrow, wmat, wrow, wmat, wrow, wmat, wrow],
        out_specs=[wrow, wrow, wrow],
        out_shape=[jax.ShapeDtypeStruct((1, D), _F32)] * 3,
    )(states, p["stfc_w1"], b_("stfc_b1"), p["stfc_w2"], b_("stfc_b2"),
      bu_wd, b_("bu_b1"), su_wc, b_("su_b1"))

    g1, g2 = _sc_gather(s1p, s2p, i1, i2)

    te = 512
    rowe = pl.BlockSpec((te, D), lambda i: (i, 0))
    bonds_out, bonds2 = pl.pallas_call(
        _bond_body,
        grid=(e_count // te,),
        in_specs=[rowe, rowe, rowe, wmat, wrow, wmat, wrow, wmat,
                  wmat, wrow, wmat, wrow, wrow],
        out_specs=[rowe, rowe],
        out_shape=[jax.ShapeDtypeStruct((e_count, D), _F32)] * 2,
    )(xb, g1, g2, p["bfc_w1"], b_("bfc_b1"), p["bfc_w2"], b_("bfc_b2"),
      bu_wc, p["bu_w2"], b_("bu_b2"), p["bu_w3"], b_("bu_b3"), gconst)

    psum, pcnt = _sc_scatter(bonds2, i1, n_count)

    psum_spec = pl.BlockSpec((tn, D), lambda i: (i, 0))
    pcnt_spec = pl.BlockSpec((tn, D), lambda i: (i, 0))
    acc_spec = pl.BlockSpec((1, D), lambda i: (0, 0))
    sites_out, bsum, ssum = pl.pallas_call(
        _site_upd_body,
        grid=(n_count // tn,),
        in_specs=[psum_spec, pcnt_spec, rown, rown, wmat, wmat, wrow,
                  wmat, wrow, wmat, wrow],
        out_specs=[rown, acc_spec, acc_spec],
        out_shape=[jax.ShapeDtypeStruct((n_count, D), _F32),
                   jax.ShapeDtypeStruct((1, D), _F32),
                   jax.ShapeDtypeStruct((1, D), _F32)],
    )(psum, pcnt, sites1, xs, su_wa, su_wb, sconst,
      p["su_w2"], b_("su_b2"), p["su_w3"], b_("su_b3"))

    states_out = pl.pallas_call(
        _make_state_upd_body(float(e_count), float(n_count)),
        grid=(1,),
        in_specs=[wrow, wrow, wrow, wrow, wmat, wmat, wmat, wrow,
                  wmat, wrow, wmat, wrow],
        out_specs=wrow,
        out_shape=jax.ShapeDtypeStruct((1, D), _F32),
    )(bsum, ssum, states1, states, stu_wa, stu_wb, stu_wc, b_("stu_b1"),
      p["stu_w2"], b_("stu_b2"), p["stu_w3"], b_("stu_b3"))

    return (sites_out[None], bonds_out[None], states_out)


# speedup vs baseline: 1.8151x; 1.2070x over previous
import functools

import jax
import jax.numpy as jnp
from jax import lax
from jax.experimental import pallas as pl
from jax.experimental.pallas import tpu as pltpu
from jax.experimental.pallas import tpu_sc as plsc

D = 128
_NC, _NS = 2, 16
_NW = _NC * _NS
_CHUNK = 80

_F32 = jnp.float32


def _relu(x):
    return jnp.maximum(x, 0.0)


def _dot(a, b):
    return jnp.dot(a, b, preferred_element_type=_F32)


def _full(shape):
    return pl.BlockSpec(shape, lambda i: (0,) * len(shape))


def _site_pre_body(x_ref, w1, b1, w2, b2, wa, wb, s1_ref, pa_ref, pb_ref):
    x = x_ref[...]
    h = _relu(_dot(x, w1[...]) + b1[...])
    s1 = _relu(_dot(h, w2[...]) + b2[...])
    s1_ref[...] = s1
    pa_ref[...] = _dot(s1, wa[...])
    pb_ref[...] = _dot(s1, wb[...])


def _state_pre_body(st, w1, b1, w2, b2, wd, bub1, scw, sub1,
                    s1_ref, g_ref, s_ref):
    s = st[...]
    h = _relu(_dot(s, w1[...]) + b1[...])
    s1 = _relu(_dot(h, w2[...]) + b2[...])
    s1_ref[...] = s1
    g_ref[...] = _dot(s1, wd[...]) + bub1[...]
    s_ref[...] = _dot(s1, scw[...]) + sub1[...]


def _bond_body(x_ref, g1_ref, g2_ref, w1, b1, w2, b2, wc, uw2, ub2, uw3, ub3,
               gconst, bo_ref, b2o_ref):
    x = x_ref[...]
    h = _relu(_dot(x, w1[...]) + b1[...])
    b1t = _relu(_dot(h, w2[...]) + b2[...])
    h = _relu(g1_ref[...] + g2_ref[...] + _dot(b1t, wc[...]) + gconst[...])
    h = _relu(_dot(h, uw2[...]) + ub2[...])
    b2v = _dot(h, uw3[...]) + ub3[...]
    bo_ref[...] = x + b2v
    b2o_ref[...] = b2v


def _site_upd_body(ps0_ref, ps1_ref, pc0_ref, pc1_ref, s1_ref, x_ref,
                   wa, wb, sconst, w2, b2, w3, b3, out_ref, bp_ref, sp_ref):
    i = pl.program_id(0)
    sums = ps0_ref[...] + ps1_ref[...]
    cnt = pc0_ref[:, 0:1] + pc1_ref[:, 0:1]
    pool = sums / jnp.maximum(cnt, 1.0)
    h = _relu(_dot(pool, wa[...]) + _dot(s1_ref[...], wb[...]) + sconst[...])
    h = _relu(_dot(h, w2[...]) + b2[...])
    s2 = _relu(_dot(h, w3[...]) + b3[...])
    out_ref[...] = x_ref[...] + s2

    @pl.when(i == 0)
    def _init():
        bp_ref[...] = jnp.zeros_like(bp_ref)
        sp_ref[...] = jnp.zeros_like(sp_ref)

    bp_ref[...] += jnp.sum(sums, axis=0, keepdims=True)
    sp_ref[...] += jnp.sum(s2, axis=0, keepdims=True)


def _make_state_upd_body(e_count, n_count):
    def _state_upd_body(bp, sp, s1, st0, ta, tb, tcw, b1, w2, b2, w3, b3,
                        out_ref):
        bpv = bp[...] * (1.0 / e_count)
        spv = sp[...] * (1.0 / n_count)
        h = _relu(_dot(bpv, ta[...]) + _dot(spv, tb[...])
                  + _dot(s1[...], tcw[...]) + b1[...])
        h = _relu(_dot(h, w2[...]) + b2[...])
        s2 = _relu(_dot(h, w3[...]) + b3[...])
        out_ref[...] = st0[...] + s2
    return _state_upd_body


def _sc_gather(t1, t2, i1, i2):
    e_count = i1.shape[0]
    epw = e_count // _NW
    nit = epw // _CHUNK
    mesh = plsc.VectorSubcoreMesh(core_axis_name="c", subcore_axis_name="s")

    @functools.partial(
        pl.kernel, mesh=mesh,
        out_type=(jax.ShapeDtypeStruct((e_count, D), _F32),
                  jax.ShapeDtypeStruct((e_count, D), _F32)),
        scratch_types=[
            pltpu.VMEM((_CHUNK,), jnp.int32),
            pltpu.VMEM((_CHUNK,), jnp.int32),
            pltpu.VMEM((_CHUNK, D), _F32),
            pltpu.VMEM((_CHUNK, D), _F32),
            pltpu.SemaphoreType.DMA,
            pltpu.SemaphoreType.DMA,
        ],
    )
    def gather_k(t1h, t2h, i1h, i2h, g1h, g2h, i1v, i2v, r1v, r2v, sm1, sm2):
        wid = lax.axis_index("s") * _NC + lax.axis_index("c")
        base = wid * epw

        def body(j, carry):
            off = base + j * _CHUNK
            pltpu.sync_copy(i1h.at[pl.ds(off, _CHUNK)], i1v)
            pltpu.sync_copy(i2h.at[pl.ds(off, _CHUNK)], i2v)
            cp1 = pltpu.async_copy(t1h.at[i1v], r1v, sm1)
            cp2 = pltpu.async_copy(t2h.at[i2v], r2v, sm2)
            cp1.wait()
            cp2.wait()
            pltpu.sync_copy(r1v, g1h.at[pl.ds(off, _CHUNK)])
            pltpu.sync_copy(r2v, g2h.at[pl.ds(off, _CHUNK)])
            return carry

        lax.fori_loop(0, nit, body, 0)

    return gather_k(t1, t2, i1, i2)


def _sc_scatter(b2, i1, n_count):
    e_count = i1.shape[0]
    epw = e_count // _NW
    nit = epw // _CHUNK
    align = _NS * _CHUNK
    npad = ((n_count + align - 1) // align) * align
    rps = npad // _NS
    fpc = rps // _CHUNK
    mesh = plsc.VectorSubcoreMesh(core_axis_name="c", subcore_axis_name="s")
    zd = jnp.zeros((_CHUNK, D), _F32)
    ones = jnp.ones((_CHUNK, D), _F32)

    @functools.partial(
        pl.kernel, mesh=mesh,
        out_type=(jax.ShapeDtypeStruct((2 * npad, D), _F32),
                  jax.ShapeDtypeStruct((2 * npad, D), _F32)),
        scratch_types=[
            pltpu.VMEM((_CHUNK,), jnp.int32),
            pltpu.VMEM((_CHUNK, D), _F32),
            pltpu.VMEM((_CHUNK, D), _F32),
            pltpu.VMEM((_CHUNK, D), _F32),
            pltpu.VMEM_SHARED((npad, D), _F32),
        ],
    )
    def scatter_k(b2h, i1h, zdh, onesh, osum, ocnt,
                  i1v, rv, zbuf, ones_v, ssum):
        sid = lax.axis_index("s")
        cid = lax.axis_index("c")
        pltpu.sync_copy(zdh, zbuf)
        pltpu.sync_copy(onesh, ones_v)
        rbase = sid * rps

        def zero():
            def zb(t, carry):
                pltpu.sync_copy(zbuf,
                                ssum.at[pl.ds(rbase + t * _CHUNK, _CHUNK)])
                return carry

            lax.fori_loop(0, fpc, zb, 0)

        def flush(dsth):
            def fb(t, carry):
                srcr = rbase + t * _CHUNK
                pltpu.sync_copy(ssum.at[pl.ds(srcr, _CHUNK)], rv)
                pltpu.sync_copy(rv, dsth.at[pl.ds(cid * npad + srcr, _CHUNK)])
                return carry

            lax.fori_loop(0, fpc, fb, 0)

        base = (cid * _NS + sid) * epw

        zero()
        plsc.subcore_barrier()

        def body(j, carry):
            off = base + j * _CHUNK
            pltpu.sync_copy(i1h.at[pl.ds(off, _CHUNK)], i1v)
            pltpu.sync_copy(b2h.at[pl.ds(off, _CHUNK)], rv)
            pltpu.sync_copy(rv, ssum.at[i1v], add=True)
            return carry

        lax.fori_loop(0, nit, body, 0)
        plsc.subcore_barrier()
        flush(osum)
        plsc.subcore_barrier()

        zero()
        plsc.subcore_barrier()

        def cbody(j, carry):
            off = base + j * _CHUNK
            pltpu.sync_copy(i1h.at[pl.ds(off, _CHUNK)], i1v)
            pltpu.sync_copy(ones_v, ssum.at[i1v], add=True)
            return carry

        lax.fori_loop(0, nit, cbody, 0)
        plsc.subcore_barrier()
        flush(ocnt)

    return scatter_k(b2, i1, zd, ones)


def kernel(sites, bonds, states, params, indices1, indices2):
    p = params
    xs = sites[0]
    xb = bonds[0]
    n_count, e_count = xs.shape[0], xb.shape[0]
    i1 = indices1.astype(jnp.int32)
    i2 = indices2.astype(jnp.int32)

    def b_(name):
        return p[name].reshape(1, D)

    wmat = _full((D, D))
    wrow = _full((1, D))
    bu_wa = p["bu_w1"][0:D]
    bu_wb = p["bu_w1"][D:2 * D]
    bu_wc = p["bu_w1"][2 * D:3 * D]
    bu_wd = p["bu_w1"][3 * D:]
    su_wa = p["su_w1"][0:D]
    su_wb = p["su_w1"][D:2 * D]
    su_wc = p["su_w1"][2 * D:]
    stu_wa = p["stu_w1"][0:D]
    stu_wb = p["stu_w1"][D:2 * D]
    stu_wc = p["stu_w1"][2 * D:]

    tn = 1000
    rown = pl.BlockSpec((tn, D), lambda i: (i, 0))
    sites1, s1p, s2p = pl.pallas_call(
        _site_pre_body,
        grid=(n_count // tn,),
        in_specs=[rown, wmat, wrow, wmat, wrow, wmat, wmat],
        out_specs=[rown, rown, rown],
        out_shape=[jax.ShapeDtypeStruct((n_count, D), _F32)] * 3,
    )(xs, p["sfc_w1"], b_("sfc_b1"), p["sfc_w2"], b_("sfc_b2"), bu_wa, bu_wb)

    states1, gconst, sconst = pl.pallas_call(
        _state_pre_body,
        grid=(1,),
        in_specs=[wrow, wmat, wrow, wmat, wrow, wmat, wrow, wmat, wrow],
        out_specs=[wrow, wrow, wrow],
        out_shape=[jax.ShapeDtypeStruct((1, D), _F32)] * 3,
    )(states, p["stfc_w1"], b_("stfc_b1"), p["stfc_w2"], b_("stfc_b2"),
      bu_wd, b_("bu_b1"), su_wc, b_("su_b1"))

    g1, g2 = _sc_gather(s1p, s2p, i1, i2)

    te = 512
    rowe = pl.BlockSpec((te, D), lambda i: (i, 0))
    bonds_out, bonds2 = pl.pallas_call(
        _bond_body,
        grid=(e_count // te,),
        in_specs=[rowe, rowe, rowe, wmat, wrow, wmat, wrow, wmat,
                  wmat, wrow, wmat, wrow, wrow],
        out_specs=[rowe, rowe],
        out_shape=[jax.ShapeDtypeStruct((e_count, D), _F32)] * 2,
    )(xb, g1, g2, p["bfc_w1"], b_("bfc_b1"), p["bfc_w2"], b_("bfc_b2"),
      bu_wc, p["bu_w2"], b_("bu_b2"), p["bu_w3"], b_("bu_b3"), gconst)

    psum, pcnt = _sc_scatter(bonds2, i1, n_count)
    npad = psum.shape[0] // 2
    ps0, ps1 = psum[:npad], psum[npad:]
    pc0, pc1 = pcnt[:npad], pcnt[npad:]

    psum_spec = pl.BlockSpec((tn, D), lambda i: (i, 0))
    acc_spec = pl.BlockSpec((1, D), lambda i: (0, 0))
    sites_out, bsum, ssum = pl.pallas_call(
        _site_upd_body,
        grid=(n_count // tn,),
        in_specs=[psum_spec, psum_spec, psum_spec, psum_spec, rown, rown,
                  wmat, wmat, wrow, wmat, wrow, wmat, wrow],
        out_specs=[rown, acc_spec, acc_spec],
        out_shape=[jax.ShapeDtypeStruct((n_count, D), _F32),
                   jax.ShapeDtypeStruct((1, D), _F32),
                   jax.ShapeDtypeStruct((1, D), _F32)],
    )(ps0, ps1, pc0, pc1, sites1, xs, su_wa, su_wb, sconst,
      p["su_w2"], b_("su_b2"), p["su_w3"], b_("su_b3"))

    states_out = pl.pallas_call(
        _make_state_upd_body(float(e_count), float(n_count)),
        grid=(1,),
        in_specs=[wrow, wrow, wrow, wrow, wmat, wmat, wmat, wrow,
                  wmat, wrow, wmat, wrow],
        out_specs=wrow,
        out_shape=jax.ShapeDtypeStruct((1, D), _F32),
    )(bsum, ssum, states1, states, stu_wa, stu_wb, stu_wc, b_("stu_b1"),
      p["stu_w2"], b_("stu_b2"), p["stu_w3"], b_("stu_b3"))

    return (sites_out[None], bonds_out[None], states_out)
